# Initial kernel scaffold; baseline (speedup 1.0000x reference)
import functools

import jax
import jax.numpy as jnp
from jax import lax
from jax.experimental import pallas as pl
from jax.experimental.pallas import tpu as pltpu
from jax.experimental.pallas import tpu_sc as plsc

B = 4096
L = 50
D = 64
N = B * L
NC = 2
NS = 16
NW = NC * NS
PER_W = N // NW
CHUNK = 400
NCHUNK = PER_W // CHUNK
GSIZE = 80
NGATHER = CHUNK // GSIZE
PHRASES_PER_CHUNK = CHUNK // L


def _body(phrase_hbm, table_hbm, pos_hbm, out_hbm, idx_v, pos_v, rows_v, sem):
    wid = lax.axis_index("s") * NC + lax.axis_index("c")
    base = wid * PER_W

    pltpu.sync_copy(phrase_hbm.at[pl.ds(base, PER_W)], idx_v)
    pltpu.sync_copy(pos_hbm, pos_v)

    for c in range(NCHUNK):
        cbase = c * CHUNK
        for g in range(NGATHER):
            pltpu.async_copy(
                table_hbm.at[idx_v.at[pl.ds(cbase + g * GSIZE, GSIZE)]],
                rows_v.at[pl.ds(g * GSIZE, GSIZE)],
                sem,
            )
        for g in range(NGATHER):
            pltpu.make_async_copy(
                table_hbm.at[idx_v.at[pl.ds(cbase + g * GSIZE, GSIZE)]],
                rows_v.at[pl.ds(g * GSIZE, GSIZE)],
                sem,
            ).wait()

        def add_l(l, _):
            pv0 = pos_v[l, pl.ds(0, 16)]
            pv1 = pos_v[l, pl.ds(16, 16)]
            pv2 = pos_v[l, pl.ds(32, 16)]
            pv3 = pos_v[l, pl.ds(48, 16)]

            def add_p(p, _):
                r = p * L + l
                plsc.addupdate(rows_v.at[r, pl.ds(0, 16)], pv0)
                plsc.addupdate(rows_v.at[r, pl.ds(16, 16)], pv1)
                plsc.addupdate(rows_v.at[r, pl.ds(32, 16)], pv2)
                plsc.addupdate(rows_v.at[r, pl.ds(48, 16)], pv3)
                return 0

            lax.fori_loop(0, PHRASES_PER_CHUNK, add_p, 0)
            return 0

        lax.fori_loop(0, L, add_l, 0)

        pltpu.sync_copy(rows_v, out_hbm.at[pl.ds(base + cbase, CHUNK)])


_sc_call = pl.kernel(
    _body,
    out_type=jax.ShapeDtypeStruct((N, D), jnp.float32),
    mesh=plsc.VectorSubcoreMesh(core_axis_name="c", subcore_axis_name="s"),
    scratch_types=[
        pltpu.VMEM((PER_W,), jnp.int32),
        pltpu.VMEM((L, D), jnp.float32),
        pltpu.VMEM((CHUNK, D), jnp.float32),
        pltpu.SemaphoreType.DMA,
    ],
)


@jax.jit
def kernel(phrase, phrase_emb, pos_emb):
    flat = phrase.reshape(N)
    out = _sc_call(flat, phrase_emb, pos_emb[:L])
    return out.reshape(B, L, D)


# speedup vs baseline: 2.0687x; 2.0687x over previous
import functools

import jax
import jax.numpy as jnp
from jax import lax
from jax.experimental import pallas as pl
from jax.experimental.pallas import tpu as pltpu
from jax.experimental.pallas import tpu_sc as plsc

B = 4096
L = 50
D = 64
N = B * L
NC = 2
NS = 16
NW = NC * NS
PER_W = N // NW
CHUNK = 400
NCHUNK = PER_W // CHUNK
GSIZE = 80
NGATHER = CHUNK // GSIZE
PHRASES_PER_CHUNK = CHUNK // L


def _body(phrase_hbm, table_hbm, pos_hbm, out_hbm, idx_v, pos_v, rows_v, sem):
    wid = lax.axis_index("s") * NC + lax.axis_index("c")
    base = wid * PER_W

    pltpu.sync_copy(phrase_hbm.at[pl.ds(base, PER_W)], idx_v)
    pltpu.sync_copy(pos_hbm, pos_v)

    for c in range(NCHUNK):
        cbase = c * CHUNK
        for g in range(NGATHER):
            pltpu.async_copy(
                table_hbm.at[idx_v.at[pl.ds(cbase + g * GSIZE, GSIZE)]],
                rows_v.at[pl.ds(g * GSIZE, GSIZE)],
                sem,
            )
        for g in range(NGATHER):
            pltpu.make_async_copy(
                table_hbm.at[idx_v.at[pl.ds(cbase + g * GSIZE, GSIZE)]],
                rows_v.at[pl.ds(g * GSIZE, GSIZE)],
                sem,
            ).wait()

        def add_l(l, _):
            pv0 = pos_v[l, pl.ds(0, 16)]
            pv1 = pos_v[l, pl.ds(16, 16)]
            pv2 = pos_v[l, pl.ds(32, 16)]
            pv3 = pos_v[l, pl.ds(48, 16)]

            def add_p(p, _):
                r = p * L + l
                plsc.addupdate(rows_v.at[r, pl.ds(0, 16)], pv0)
                plsc.addupdate(rows_v.at[r, pl.ds(16, 16)], pv1)
                plsc.addupdate(rows_v.at[r, pl.ds(32, 16)], pv2)
                plsc.addupdate(rows_v.at[r, pl.ds(48, 16)], pv3)
                return 0

            lax.fori_loop(0, PHRASES_PER_CHUNK, add_p, 0)
            return 0

        lax.fori_loop(0, L, add_l, 0)

        pltpu.sync_copy(rows_v, out_hbm.at[pl.ds(base + cbase, CHUNK)])


_sc_call = pl.kernel(
    _body,
    out_type=jax.ShapeDtypeStruct((N, D), jnp.float32),
    mesh=plsc.VectorSubcoreMesh(core_axis_name="c", subcore_axis_name="s"),
    scratch_types=[
        pltpu.VMEM((PER_W,), jnp.int32),
        pltpu.VMEM((L, D), jnp.float32),
        pltpu.VMEM((CHUNK, D), jnp.float32),
        pltpu.SemaphoreType.DMA,
    ],
    compiler_params=pltpu.CompilerParams(use_tc_tiling_on_sc=False),
)


@jax.jit
def kernel(phrase, phrase_emb, pos_emb):
    flat = phrase.reshape(N)
    out = _sc_call(flat, phrase_emb, pos_emb[:L])
    return out.reshape(B, L, D)


# speedup vs baseline: 2.1301x; 1.0297x over previous
import functools

import jax
import jax.numpy as jnp
from jax import lax
from jax.experimental import pallas as pl
from jax.experimental.pallas import tpu as pltpu
from jax.experimental.pallas import tpu_sc as plsc

B = 4096
L = 50
D = 64
N = B * L
NC = 2
NS = 16
NW = NC * NS
PER_W = N // NW
CHUNK = 800
NCHUNK = PER_W // CHUNK
PHRASES = CHUNK // L
GS = (128, 128, 128, 128, 128, 128, 32)
assert sum(GS) == CHUNK


def _body(phrase_hbm, table_hbm, pos_hbm, out_hbm,
          idx_v, pos_v, rows0, rows1, gsem0, gsem1, ssem0, ssem1):
    wid = lax.axis_index("s") * NC + lax.axis_index("c")
    base = wid * PER_W

    bufs = (rows0, rows1)
    gsems = (gsem0, gsem1)
    ssems = (ssem0, ssem1)

    pltpu.sync_copy(phrase_hbm.at[pl.ds(base, PER_W)], idx_v)
    pltpu.sync_copy(pos_hbm, pos_v)

    def fire_gathers(c):
        buf, sem = bufs[c % 2], gsems[c % 2]
        off = 0
        for g in GS:
            pltpu.async_copy(
                table_hbm.at[idx_v.at[pl.ds(c * CHUNK + off, g)]],
                buf.at[pl.ds(off, g)],
                sem,
            )
            off += g

    def drain_gathers(c):
        buf, sem = bufs[c % 2], gsems[c % 2]
        off = 0
        for g in GS:
            pltpu.make_async_copy(
                table_hbm.at[idx_v.at[pl.ds(c * CHUNK + off, g)]],
                buf.at[pl.ds(off, g)],
                sem,
            ).wait()
            off += g

    def add_pos(c):
        buf = bufs[c % 2]

        def add_l(l, _):
            pv0 = pos_v[l, pl.ds(0, 16)]
            pv1 = pos_v[l, pl.ds(16, 16)]
            pv2 = pos_v[l, pl.ds(32, 16)]
            pv3 = pos_v[l, pl.ds(48, 16)]

            def add_p(p, _):
                r = p * L + l
                plsc.addupdate(buf.at[r, pl.ds(0, 16)], pv0)
                plsc.addupdate(buf.at[r, pl.ds(16, 16)], pv1)
                plsc.addupdate(buf.at[r, pl.ds(32, 16)], pv2)
                plsc.addupdate(buf.at[r, pl.ds(48, 16)], pv3)
                return 0

            lax.fori_loop(0, PHRASES, add_p, 0)
            return 0

        lax.fori_loop(0, L, add_l, 0)

    def fire_scatter(c):
        buf, sem = bufs[c % 2], ssems[c % 2]
        pltpu.async_copy(buf, out_hbm.at[pl.ds(base + c * CHUNK, CHUNK)], sem)

    def wait_scatter(c):
        buf, sem = bufs[c % 2], ssems[c % 2]
        pltpu.make_async_copy(
            buf, out_hbm.at[pl.ds(base + c * CHUNK, CHUNK)], sem
        ).wait()

    for c in range(NCHUNK):
        if c >= 2:
            wait_scatter(c - 2)
        fire_gathers(c)
        if c >= 1:
            drain_gathers(c - 1)
            add_pos(c - 1)
            fire_scatter(c - 1)

    drain_gathers(NCHUNK - 1)
    add_pos(NCHUNK - 1)
    fire_scatter(NCHUNK - 1)
    wait_scatter(NCHUNK - 2)
    wait_scatter(NCHUNK - 1)


_sc_call = pl.kernel(
    _body,
    out_type=jax.ShapeDtypeStruct((N, D), jnp.float32),
    mesh=plsc.VectorSubcoreMesh(core_axis_name="c", subcore_axis_name="s"),
    scratch_types=[
        pltpu.VMEM((PER_W,), jnp.int32),
        pltpu.VMEM((L, D), jnp.float32),
        pltpu.VMEM((CHUNK, D), jnp.float32),
        pltpu.VMEM((CHUNK, D), jnp.float32),
        pltpu.SemaphoreType.DMA,
        pltpu.SemaphoreType.DMA,
        pltpu.SemaphoreType.DMA,
        pltpu.SemaphoreType.DMA,
    ],
    compiler_params=pltpu.CompilerParams(use_tc_tiling_on_sc=False),
)


@jax.jit
def kernel(phrase, phrase_emb, pos_emb):
    flat = phrase.reshape(N)
    out = _sc_call(flat, phrase_emb, pos_emb[:L])
    return out.reshape(B, L, D)
